# baseline (speedup 1.0000x reference)
import functools
import jax
import jax.numpy as jnp
from jax import lax
from jax.experimental import pallas as pl
from jax.experimental.pallas import tpu as pltpu
from jax.experimental.pallas import tpu_sc as plsc

X_ID = 343
NBUF = 16
LANES = 16


def _lane_extract(vec, lane):
    lanes = lax.iota(jnp.int32, LANES)
    return jnp.max(jnp.where(lanes == lane, vec, -1))


def _make_sc_kernel(b, t, d, n_ctx, dtype):
    info = plsc.get_sparse_core_info()
    nc, ns = info.num_cores, info.num_subcores
    nw = nc * ns
    s_per_w = b // nw
    n_chunks = s_per_w // LANES
    mesh = plsc.VectorSubcoreMesh(core_axis_name="c", subcore_axis_name="s")

    @functools.partial(
        pl.kernel,
        out_type=jax.ShapeDtypeStruct((b, t, d), dtype),
        scratch_types=[
            pltpu.VMEM((2, t, d), dtype),
            pltpu.VMEM((s_per_w,), jnp.int32),
            pltpu.VMEM((LANES,), jnp.int32),
            pltpu.SemaphoreType.DMA((NBUF,)),
        ],
        mesh=mesh,
        compiler_params=pltpu.CompilerParams(needs_layout_passes=False, use_tc_tiling_on_sc=False),
    )
    def sc_kernel(vl_hbm, pr_hbm, tps_hbm, tv_hbm, xpos_hbm, out_hbm,
                  t_v, lbl_v, xpos_v, sems):
        wid = lax.axis_index("s") * nc + lax.axis_index("c")
        base = wid * s_per_w
        pltpu.sync_copy(tps_hbm, t_v.at[0])
        pltpu.sync_copy(tps_hbm, t_v.at[1])
        pltpu.sync_copy(xpos_hbm, xpos_v)
        pltpu.sync_copy(vl_hbm.at[pl.ds(base, s_per_w)], lbl_v)
        xpos_vec = xpos_v[...]
        for j in range(n_ctx):
            p = _lane_extract(xpos_vec, j)
            pltpu.sync_copy(pr_hbm.at[j], t_v.at[0, p])
            pltpu.sync_copy(pr_hbm.at[j], t_v.at[1, p])
        pv = _lane_extract(xpos_vec, n_ctx)
        pltpu.sync_copy(tv_hbm.at[0], t_v.at[0, pv])
        pltpu.sync_copy(tv_hbm.at[1], t_v.at[1, pv])

        def _dma(i, lbl):
            return pltpu.make_async_copy(
                t_v.at[lbl], out_hbm.at[base + i], sems.at[lax.rem(i, NBUF)]
            )

        def chunk_body(c, carry):
            vec = lbl_v[pl.ds(pl.multiple_of(c * LANES, LANES), LANES)]
            for l in range(LANES):
                i = c * LANES + l
                @pl.when(i >= NBUF)
                def _():
                    _dma(i - NBUF, 0).wait()
                lbl = _lane_extract(vec, l)
                _dma(i, lbl).start()
            return carry

        lax.fori_loop(0, n_chunks, chunk_body, 0)
        for k in range(NBUF):
            _dma(s_per_w - NBUF + k, 0).wait()

    return sc_kernel


def kernel(view_label, prompts, token_prefix_suffix, token_view, tokenized_prompts):
    b = view_label.shape[0]
    t, d = token_prefix_suffix.shape[1], token_prefix_suffix.shape[2]
    n_ctx = prompts.shape[1]
    vl = view_label.astype(jnp.int32)
    pr = prompts.reshape(n_ctx, d)
    tps = token_prefix_suffix.reshape(t, d)
    tv = token_view[0, 1:3, :]
    x_pos = jnp.nonzero(tokenized_prompts == X_ID, size=n_ctx + 1)[1]
    xpos = jnp.zeros((LANES,), jnp.int32).at[: n_ctx + 1].set(x_pos.astype(jnp.int32))
    sc = _make_sc_kernel(b, t, d, n_ctx, token_prefix_suffix.dtype)
    return sc(vl, pr, tps, tv, xpos)


# speedup vs baseline: 1.0373x; 1.0373x over previous
import jax
import jax.numpy as jnp
from jax.experimental import pallas as pl
from jax.experimental.pallas import tpu as pltpu

X_ID = 343
NBUF = 8
GROUP = 8
HALF = 4


def _copy_kernel(codes_smem, tok_ref, prompts_ref, tps_ref, tv_ref, out_hbm,
                 buf_a, buf_b, sems):
    t = tok_ref.shape[1]
    rows = GROUP * t
    half_rows = HALF * t
    n_groups = out_hbm.shape[0] // rows
    n_ctx = prompts_ref.shape[1]
    tok_row = tok_ref[...]
    xm_row = (tok_row == X_ID).astype(jnp.int32)
    r = jax.lax.broadcasted_iota(jnp.int32, (t, t), 0)
    c = jax.lax.broadcasted_iota(jnp.int32, (t, t), 1)
    cnt_incl = jnp.sum(jnp.where(c <= r, xm_row, 0), axis=1, keepdims=True)
    cnt_excl = jnp.sum(jnp.where(c < r, xm_row, 0), axis=1, keepdims=True)
    xm = (cnt_incl - cnt_excl) > 0
    cnt = cnt_incl - 1
    base = tps_ref[0]
    for j in range(n_ctx):
        base = jnp.where(xm & (cnt == j), prompts_ref[0, j][None, :], base)
    view_slot = xm & (cnt == n_ctx)
    tm = [jnp.where(view_slot, tv_ref[0, 1][None, :], base),
          jnp.where(view_slot, tv_ref[0, 2][None, :], base)]
    for m in range(2 ** (HALF + 1)):
        for k in range(HALF):
            buf_a[m, pl.ds(k * t, t)] = tm[(m >> k) & 1]
        buf_a[m, pl.ds(HALF * t, 4)] = tm[(m >> HALF) & 1][:4, :]
    for m in range(2 ** (HALF + 1)):
        buf_b[m, pl.ds(0, 4)] = tm[m & 1][t - 4:, :]
        for k in range(HALF):
            buf_b[m, pl.ds(4 + k * t, t)] = tm[(m >> (k + 1)) & 1]

    def _dma_a(i):
        return pltpu.make_async_copy(
            buf_a.at[codes_smem[2 * i]],
            out_hbm.at[pl.ds(i * rows, half_rows + 4)],
            sems.at[jax.lax.rem(2 * i, NBUF)],
        )

    def _dma_b(i):
        return pltpu.make_async_copy(
            buf_b.at[codes_smem[2 * i + 1]],
            out_hbm.at[pl.ds(i * rows + half_rows - 4, half_rows + 4)],
            sems.at[jax.lax.rem(2 * i + 1, NBUF)],
        )

    def body(i, _):
        @pl.when(i >= NBUF // 2)
        def _():
            _dma_a(i - NBUF // 2).wait()
            _dma_b(i - NBUF // 2).wait()
        _dma_a(i).start()
        _dma_b(i).start()
        return 0

    jax.lax.fori_loop(0, n_groups, body, 0)
    for k in range(NBUF // 2):
        _dma_a(n_groups - NBUF // 2 + k).wait()
        _dma_b(n_groups - NBUF // 2 + k).wait()


def kernel(view_label, prompts, token_prefix_suffix, token_view, tokenized_prompts):
    b = view_label.shape[0]
    t, d = token_prefix_suffix.shape[1], token_prefix_suffix.shape[2]
    tok = tokenized_prompts.astype(jnp.int32).reshape(1, t)
    vl = view_label.astype(jnp.int32).reshape(b // GROUP, GROUP)
    w = jnp.asarray([1 << k for k in range(HALF)], dtype=jnp.int32)
    code_a = vl[:, :HALF] @ w + (vl[:, HALF] << HALF)
    code_b = vl[:, HALF - 1] + 2 * (vl[:, HALF:] @ w)
    codes = jnp.stack([code_a, code_b], axis=1).reshape(-1)
    grid_spec = pltpu.PrefetchScalarGridSpec(
        num_scalar_prefetch=1,
        grid=(1,),
        in_specs=[
            pl.BlockSpec((1, t), lambda i, c_ref: (0, 0)),
            pl.BlockSpec((1, prompts.shape[1], d), lambda i, c_ref: (0, 0, 0)),
            pl.BlockSpec((1, t, d), lambda i, c_ref: (0, 0, 0)),
            pl.BlockSpec((1, t, d), lambda i, c_ref: (0, 0, 0)),
        ],
        out_specs=pl.BlockSpec(memory_space=pl.ANY),
        scratch_shapes=[
            pltpu.VMEM((2 ** (HALF + 1), HALF * t + 4, d), token_prefix_suffix.dtype),
            pltpu.VMEM((2 ** (HALF + 1), HALF * t + 4, d), token_prefix_suffix.dtype),
            pltpu.SemaphoreType.DMA((NBUF,)),
        ],
    )
    out2d = pl.pallas_call(
        _copy_kernel,
        grid_spec=grid_spec,
        out_shape=jax.ShapeDtypeStruct((b * t, d), token_prefix_suffix.dtype),
    )(codes, tok, prompts, token_prefix_suffix, token_view)
    return out2d.reshape(b, t, d)


# speedup vs baseline: 2.0597x; 1.9857x over previous
import jax
import jax.numpy as jnp
from jax.experimental import pallas as pl
from jax.experimental.pallas import tpu as pltpu

X_ID = 343
NBUF = 32
GROUP = 4


def _copy_kernel(codes_smem, tok_ref, prompts_ref, tps_ref, tv_ref, out_hbm,
                 buf_v, sems):
    n_groups = out_hbm.shape[0] // GROUP
    t = tok_ref.shape[1]
    n_ctx = prompts_ref.shape[1]
    tok_row = tok_ref[...]
    xm_row = (tok_row == X_ID).astype(jnp.int32)
    r = jax.lax.broadcasted_iota(jnp.int32, (t, t), 0)
    c = jax.lax.broadcasted_iota(jnp.int32, (t, t), 1)
    cnt_incl = jnp.sum(jnp.where(c <= r, xm_row, 0), axis=1, keepdims=True)
    cnt_excl = jnp.sum(jnp.where(c < r, xm_row, 0), axis=1, keepdims=True)
    xm = (cnt_incl - cnt_excl) > 0
    cnt = cnt_incl - 1
    base = tps_ref[0]
    for j in range(n_ctx):
        base = jnp.where(xm & (cnt == j), prompts_ref[0, j][None, :], base)
    view_slot = xm & (cnt == n_ctx)
    tmpl0 = jnp.where(view_slot, tv_ref[0, 1][None, :], base)
    tmpl1 = jnp.where(view_slot, tv_ref[0, 2][None, :], base)
    for q in range(2 ** GROUP):
        for k in range(GROUP):
            buf_v[q, k] = tmpl1 if (q >> k) & 1 else tmpl0

    def _dma(i):
        return pltpu.make_async_copy(
            buf_v.at[codes_smem[i]],
            out_hbm.at[pl.ds(i * GROUP, GROUP)],
            sems.at[jax.lax.rem(i, NBUF)],
        )

    def body(i, _):
        @pl.when(i >= NBUF)
        def _():
            _dma(i - NBUF).wait()
        _dma(i).start()
        return 0

    jax.lax.fori_loop(0, n_groups, body, 0)
    for k in range(NBUF):
        _dma(n_groups - NBUF + k).wait()


def kernel(view_label, prompts, token_prefix_suffix, token_view, tokenized_prompts):
    b = view_label.shape[0]
    t, d = token_prefix_suffix.shape[1], token_prefix_suffix.shape[2]
    tok = tokenized_prompts.astype(jnp.int32).reshape(1, t)
    vl = view_label.astype(jnp.int32).reshape(b // GROUP, GROUP)
    codes = vl @ jnp.asarray([1 << k for k in range(GROUP)], dtype=jnp.int32)
    grid_spec = pltpu.PrefetchScalarGridSpec(
        num_scalar_prefetch=1,
        grid=(1,),
        in_specs=[
            pl.BlockSpec((1, t), lambda i, c_ref: (0, 0)),
            pl.BlockSpec((1, prompts.shape[1], d), lambda i, c_ref: (0, 0, 0)),
            pl.BlockSpec((1, t, d), lambda i, c_ref: (0, 0, 0)),
            pl.BlockSpec((1, t, d), lambda i, c_ref: (0, 0, 0)),
        ],
        out_specs=pl.BlockSpec(memory_space=pl.ANY),
        scratch_shapes=[
            pltpu.VMEM((2 ** GROUP, GROUP, t, d), token_prefix_suffix.dtype),
            pltpu.SemaphoreType.DMA((NBUF,)),
        ],
    )
    return pl.pallas_call(
        _copy_kernel,
        grid_spec=grid_spec,
        out_shape=jax.ShapeDtypeStruct((b, t, d), token_prefix_suffix.dtype),
    )(codes, tok, prompts, token_prefix_suffix, token_view)
